# Initial kernel scaffold; baseline (speedup 1.0000x reference)
import jax
import jax.numpy as jnp
from jax.experimental import pallas as pl


def kernel(x, edge_index, W1l, b1, W1r, W2l, b2, W2r, W3, b3, W4, b4):
    raise NotImplementedError("write your pallas kernel here")


# speedup vs baseline: 5.3311x; 5.3311x over previous
import functools

import jax
import jax.numpy as jnp
from jax import lax
from jax.experimental import pallas as pl
from jax.experimental.pallas import tpu as pltpu
from jax.experimental.pallas import tpu_sc as plsc

N = 10000
E = 320000
D = 128

NC = 2
NS = 16
NW = NC * NS
EPW = E // NW
C = 80
NCHUNK = EPW // C
RPT = N // NS
ZR = 125
CW = 16


def _make_seg_sum(with_counts):
    mesh = plsc.VectorSubcoreMesh(
        core_axis_name="c", subcore_axis_name="s",
        num_cores=NC, num_subcores=NS)

    out_type = [jax.ShapeDtypeStruct((NC * N, D), jnp.float32)]
    scratch = [
        pltpu.VMEM((C,), jnp.int32),
        pltpu.VMEM((C,), jnp.int32),
        pltpu.VMEM((C, D), jnp.float32),
        pltpu.VMEM((ZR, D), jnp.float32),
    ]
    if with_counts:
        out_type.append(jax.ShapeDtypeStruct((NC * N, CW), jnp.float32))
        scratch += [
            pltpu.VMEM((C, CW), jnp.float32),
            pltpu.VMEM((ZR, CW), jnp.float32),
        ]
    scratch.append(pltpu.VMEM_SHARED((N, D), jnp.float32))
    if with_counts:
        scratch.append(pltpu.VMEM_SHARED((N, CW), jnp.float32))
    scratch.append(pltpu.SemaphoreType.DMA)

    def body(*refs):
        if with_counts:
            (table, src, dst, sums_out, cnts_out,
             idx_s, idx_d, rows, zrows, ones, zc, acc, cacc, sem) = refs
        else:
            (table, src, dst, sums_out,
             idx_s, idx_d, rows, zrows, acc, sem) = refs

        core = lax.axis_index("c")
        sid = lax.axis_index("s")
        gw = core * NS + sid

        def zfill(i, _):
            for j in range(D // 16):
                zrows[i, pl.ds(j * 16, 16)] = jnp.zeros((16,), jnp.float32)
            return 0
        lax.fori_loop(0, ZR, zfill, 0)
        if with_counts:
            def zcfill(i, _):
                zc[i, pl.ds(0, CW)] = jnp.zeros((CW,), jnp.float32)
                return 0
            lax.fori_loop(0, ZR, zcfill, 0)

            def onesfill(i, _):
                ones[i, pl.ds(0, CW)] = jnp.ones((CW,), jnp.float32)
                return 0
            lax.fori_loop(0, C, onesfill, 0)

        r0 = sid * RPT
        for i in range(RPT // ZR):
            pltpu.sync_copy(zrows, acc.at[pl.ds(r0 + i * ZR, ZR), :])
            if with_counts:
                pltpu.sync_copy(zc, cacc.at[pl.ds(r0 + i * ZR, ZR), :])
        plsc.subcore_barrier()

        base = gw * EPW

        def edge_step(i, _):
            off = base + i * C
            pltpu.sync_copy(src.at[pl.ds(off, C)], idx_s)
            pltpu.sync_copy(dst.at[pl.ds(off, C)], idx_d)
            pltpu.async_copy(table.at[idx_s], rows, sem).wait()
            pltpu.sync_copy(rows, acc.at[idx_d], add=True)
            if with_counts:
                pltpu.sync_copy(ones, cacc.at[idx_d], add=True)
            return 0
        lax.fori_loop(0, NCHUNK, edge_step, 0)

        plsc.subcore_barrier()

        out0 = core * N + r0
        for i in range(RPT // ZR):
            pltpu.sync_copy(acc.at[pl.ds(r0 + i * ZR, ZR), :], zrows)
            pltpu.sync_copy(zrows, sums_out.at[pl.ds(out0 + i * ZR, ZR), :])
            if with_counts:
                pltpu.sync_copy(cacc.at[pl.ds(r0 + i * ZR, ZR), :], zc)
                pltpu.sync_copy(zc, cnts_out.at[pl.ds(out0 + i * ZR, ZR), :])

    return pl.kernel(body, out_type=out_type, mesh=mesh,
                     scratch_types=scratch,
                     compiler_params=pltpu.CompilerParams(
                         use_tc_tiling_on_sc=False))


_seg_sum_counts = _make_seg_sum(True)
_seg_sum = _make_seg_sum(False)


NB = 400
NGRID = N // NB


def _tc_layer1(sa_ref, sb_ref, ca_ref, cb_ref, x_ref, wl_ref, b_ref, wr_ref,
               out_ref):
    s = sa_ref[...] + sb_ref[...]
    c = ca_ref[...] + cb_ref[...]
    cnt = c[:, 0:1]
    agg = s * (1.0 / jnp.maximum(cnt, 1.0))
    h = (jnp.dot(agg, wl_ref[...], preferred_element_type=jnp.float32)
         + b_ref[...]
         + jnp.dot(x_ref[...], wr_ref[...], preferred_element_type=jnp.float32))
    out_ref[...] = jnp.maximum(h, 0.0)


def _tc_layer2(sa_ref, sb_ref, ca_ref, cb_ref, h1_ref, wl_ref, b2_ref, wr_ref,
               w3_ref, b3_ref, w4_ref, b4_ref, out_ref):
    s = sa_ref[...] + sb_ref[...]
    c = ca_ref[...] + cb_ref[...]
    cnt = c[:, 0:1]
    agg = s * (1.0 / jnp.maximum(cnt, 1.0))
    h = (jnp.dot(agg, wl_ref[...], preferred_element_type=jnp.float32)
         + b2_ref[...]
         + jnp.dot(h1_ref[...], wr_ref[...], preferred_element_type=jnp.float32))
    h = jnp.maximum(h, 0.0)
    nrm = jnp.sqrt(jnp.sum(h * h, axis=1, keepdims=True))
    h = h / jnp.maximum(nrm, 1e-12)
    hid = jnp.maximum(
        jnp.dot(h, w3_ref[...], preferred_element_type=jnp.float32)
        + b3_ref[...], 0.0)
    logits = (jnp.dot(hid, w4_ref[...], preferred_element_type=jnp.float32)
              + b4_ref[...])
    col = lax.broadcasted_iota(jnp.int32, logits.shape, 1)
    logits = jnp.where(col < 4, logits, -1e30)
    m = jnp.max(logits, axis=1, keepdims=True)
    e = jnp.exp(logits - m)
    out_ref[...] = e / jnp.sum(e, axis=1, keepdims=True)


def _row_spec(w):
    return pl.BlockSpec((NB, w), lambda i: (i, 0))


def _row_spec_hi(w):
    return pl.BlockSpec((NB, w), lambda i: (i + NGRID, 0))


def _full_spec(r, c):
    return pl.BlockSpec((r, c), lambda i: (0, 0))


def kernel(x, edge_index, W1l, b1, W1r, W2l, b2, W2r, W3, b3, W4, b4):
    src = edge_index[0]
    dst = edge_index[1]

    sums1, cnts = _seg_sum_counts(x, src, dst)

    h1 = pl.pallas_call(
        _tc_layer1,
        grid=(NGRID,),
        in_specs=[
            _row_spec(D), _row_spec_hi(D),
            _row_spec(CW), _row_spec_hi(CW),
            _row_spec(D),
            _full_spec(D, D), _full_spec(1, D), _full_spec(D, D),
        ],
        out_specs=_row_spec(D),
        out_shape=jax.ShapeDtypeStruct((N, D), jnp.float32),
    )(sums1, sums1, cnts, cnts, x,
      W1l.T, b1.reshape(1, D), W1r.T)

    (sums2,) = _seg_sum(h1, src, dst)

    W4t = jnp.zeros((64, 128), jnp.float32).at[:, :4].set(W4.T)
    b4p = jnp.zeros((1, 128), jnp.float32).at[0, :4].set(b4)

    probs_pad = pl.pallas_call(
        _tc_layer2,
        grid=(NGRID,),
        in_specs=[
            _row_spec(D), _row_spec_hi(D),
            _row_spec(CW), _row_spec_hi(CW),
            _row_spec(D),
            _full_spec(D, D), _full_spec(1, D), _full_spec(D, D),
            _full_spec(D, 64), _full_spec(1, 64),
            _full_spec(64, 128), _full_spec(1, 128),
        ],
        out_specs=_row_spec(128),
        out_shape=jax.ShapeDtypeStruct((N, 128), jnp.float32),
    )(sums2, sums2, cnts, cnts, h1,
      W2l.T, b2.reshape(1, D), W2r.T,
      W3.T, b3.reshape(1, 64), W4t, b4p)

    return probs_pad[:, :4]
